# baseline (device time: 66898 ns/iter reference)
import jax
import jax.numpy as jnp
from jax import lax
from jax.experimental import pallas as pl
from jax.experimental.pallas import tpu as pltpu

N_DEV = 16
N_TILE = 1024
N_SPLIT = 4


def kernel(x, w_mat, scale_x, scale_w):
    M_global, k_per = x.shape
    K, n = w_mat.shape
    m_per = M_global // N_DEV
    n_tiles = n // N_TILE

    def body(x_ref, w_ref, sx_ref, sw_ref, out_ref,
             x8, xfull, wtile, send_sems, recv_sems, wsems):
        my = lax.axis_index("i")

        x8[...] = x_ref[...].astype(jnp.float8_e4m3fn)

        k_chunk = K // N_SPLIT

        def start_tile(t, slot):
            for r in range(N_SPLIT):
                pltpu.make_async_copy(
                    w_ref.at[pl.ds(r * k_chunk, k_chunk),
                             pl.ds(t * N_TILE, N_TILE)],
                    wtile.at[slot, pl.ds(r * k_chunk, k_chunk), :],
                    wsems.at[slot, r],
                ).start()

        def wait_tile(t, slot):
            for r in range(N_SPLIT):
                pltpu.make_async_copy(
                    w_ref.at[pl.ds(r * k_chunk, k_chunk),
                             pl.ds(t * N_TILE, N_TILE)],
                    wtile.at[slot, pl.ds(r * k_chunk, k_chunk), :],
                    wsems.at[slot, r],
                ).wait()

        for t0 in range(2):
            start_tile(t0, t0)

        rdmas = []
        for d in range(1, N_DEV):
            j = lax.rem(my + d, N_DEV)
            rdma = pltpu.make_async_remote_copy(
                src_ref=x8.at[pl.ds(j * m_per, m_per), :],
                dst_ref=xfull.at[:, pl.ds(my * k_per, k_per)],
                send_sem=send_sems.at[d],
                recv_sem=recv_sems.at[d],
                device_id=(j,),
                device_id_type=pl.DeviceIdType.MESH,
            )
            rdma.start()
            rdmas.append(rdma)

        xfull[:, pl.ds(my * k_per, k_per)] = x8[pl.ds(my * m_per, m_per), :]

        for r in rdmas:
            r.wait_recv()

        s = sx_ref[0] * sw_ref[0]
        for t in range(n_tiles):
            slot = t % 2
            wait_tile(t, slot)
            acc = lax.dot_general(
                xfull[...], wtile[slot].astype(jnp.float8_e5m2),
                (((1,), (0,)), ((), ())),
                preferred_element_type=jnp.float32,
            )
            y = acc * s
            z = jnp.clip(y, -60.0, 60.0)
            out_ref[:, pl.ds(t * N_TILE, N_TILE)] = y / (1.0 + jnp.exp(-z))
            if t + 2 < n_tiles:
                start_tile(t + 2, slot)

        for r in rdmas:
            r.wait_send()

    return pl.pallas_call(
        body,
        out_shape=jax.ShapeDtypeStruct((m_per, n), jnp.float32),
        in_specs=[
            pl.BlockSpec(memory_space=pltpu.VMEM),
            pl.BlockSpec(memory_space=pltpu.MemorySpace.HBM),
            pl.BlockSpec(memory_space=pltpu.SMEM),
            pl.BlockSpec(memory_space=pltpu.SMEM),
        ],
        out_specs=pl.BlockSpec(memory_space=pltpu.VMEM),
        scratch_shapes=[
            pltpu.VMEM((M_global, k_per), jnp.float8_e4m3fn),
            pltpu.VMEM((m_per, K), jnp.float8_e4m3fn),
            pltpu.VMEM((2, K, N_TILE), jnp.float32),
            pltpu.SemaphoreType.DMA((N_DEV,)),
            pltpu.SemaphoreType.DMA((N_DEV,)),
            pltpu.SemaphoreType.DMA((2, N_SPLIT)),
        ],
        compiler_params=pltpu.CompilerParams(
            vmem_limit_bytes=100 * 1024 * 1024,
        ),
    )(x, w_mat, scale_x, scale_w)
